# baseline (speedup 1.0000x reference)
import numpy as np

import jax
import jax.numpy as jnp
from jax.experimental import pallas as pl

F32 = jnp.float32
N, T = 8, 16
NT = N * T
HID = 128
DT = 0.04
HP = jax.lax.Precision.HIGHEST


def _rollout_kernel(frames_ref, ch_ref, qi_ref, pos_ref, qt_ref, scale_ref,
                    invs8_ref, m_ref, e_ref, c_ref, b1m_ref, b2m_ref,
                    g1m_ref, g2m_ref, w1_ref, b1_ref,
                    w2ih_ref, bs_ref, whh_ref, out_ref):
    frames = frames_ref[...].astype(jnp.int32)
    q = jnp.concatenate([qi_ref[...], qt_ref[...]], axis=0)
    mask = (q.astype(jnp.int32) == frames).astype(F32)
    cnt = jnp.sum(mask, axis=1, keepdims=True)
    s = jnp.dot(mask, ch_ref[...], preferred_element_type=F32)

    pos = pos_ref[...]
    pooled0 = pos * cnt[:NT] - s[:NT]
    iseq0 = jnp.concatenate([pos, pooled0], axis=1)

    zeros4 = jnp.zeros((NT, 4), F32)
    cntpat = jnp.concatenate([zeros4 + 1.0, zeros4 + cnt[NT:]], axis=1)
    spat = jnp.concatenate([zeros4, s[NT:]], axis=1)

    w1 = w1_ref[...]
    b1 = b1_ref[...]
    w2ih = w2ih_ref[...]
    bs = bs_ref[...]
    whh = whh_ref[...]
    scale = scale_ref[...]
    invs8 = invs8_ref[...]
    mmat = m_ref[...]
    emat = e_ref[...]
    cmat = c_ref[...]
    b1m = b1m_ref[...]
    b2m = b2m_ref[...]
    g1m = g1m_ref[...]
    g2m = g2m_ref[...]

    def mlp(rows):
        fx = jnp.tanh(jnp.dot(rows, w1, preferred_element_type=F32) + b1)
        return jnp.dot(fx, w2ih, preferred_element_type=F32) + bs

    spow = [jnp.ones((1, 8), F32)]
    for k in range(1, T):
        spow.append(spow[-1] * scale)

    X = {}
    for i in range(T):
        xi = mlp(iseq0[i * N:] * spow[i] if i else iseq0)
        for t in range(T - i):
            X[(i, t)] = xi[t * N:(t + 1) * N]

    cnt_t = cnt[NT:]
    dmat = b1 - jnp.dot(spat, w1, preferred_element_type=F32)
    last0 = iseq0[NT - N:NT] * scale
    J = [None] * T
    LAST = [None] * T
    LAST[0] = last0
    J[0] = (jnp.dot(last0, b1m, preferred_element_type=F32)
            + cnt_t[0:N] * jnp.dot(last0, b2m, preferred_element_type=F32)
            + dmat[0:N])

    H = [None] * T
    C = [None] * T
    P = [None] * T
    for sl in range(3 * (T - 1) + T):
        for i in range(T):
            t = sl - 3 * i
            if t < 0 or t >= T:
                continue
            xt = X[(i, t)]
            g = xt if t == 0 else xt + jnp.dot(
                H[i], whh, preferred_element_type=F32)
            tg = jnp.tanh(g)
            ti = tg[:, 0:128]
            tf = tg[:, 128:256]
            tc = tg[:, 256:384]
            to = tg[:, 384:512]
            cc = (0.5 * ((ti + 1.0) * tc) if t == 0
                  else 0.5 * ((tf + 1.0) * C[i] + (ti + 1.0) * tc))
            hh = (0.5 * (to + 1.0)) * jnp.tanh(cc)
            H[i], C[i] = hh, cc

            if t == T - 1:
                ci = cnt_t[i * N:(i + 1) * N]
                if i < T - 1:
                    fx0 = jnp.tanh(
                        jnp.dot(hh, g1m, preferred_element_type=F32)
                        + ci * jnp.dot(hh, g2m, preferred_element_type=F32)
                        + J[i])
                    X[(i + 1, T - 1)] = jnp.dot(
                        fx0, w2ih, preferred_element_type=F32) + bs

                last = LAST[i]
                u = (jnp.dot(last, mmat, preferred_element_type=F32,
                             precision=HP)
                     + jnp.dot(hh, emat, preferred_element_type=F32,
                               precision=HP)) * invs8
                out_ref[i * N:(i + 1) * N, :] = u
                if i < T - 1:
                    nr = (jnp.dot(u, cmat, preferred_element_type=F32,
                                  precision=HP)
                          * cntpat[i * N:(i + 1) * N]
                          - spat[i * N:(i + 1) * N])
                    P[i] = nr
                    ln = nr * scale
                    LAST[i + 1] = ln
                    if i < T - 2:
                        cn = cnt_t[(i + 1) * N:(i + 2) * N]
                        J[i + 1] = (jnp.dot(ln, b1m,
                                            preferred_element_type=F32)
                                    + cn * jnp.dot(ln, b2m,
                                                   preferred_element_type=F32)
                                    + dmat[(i + 1) * N:(i + 2) * N])
                    nlad = T - 1 - i
                    if nlad > 1:
                        lad = jnp.concatenate(
                            [nr * spow[k] for k in range(1, nlad)], axis=0)
                        xl = mlp(lad)
                        for k in range(1, nlad):
                            X[(i + 1 + k, T - 1 - k)] = xl[(k - 1) * N:k * N]


@jax.jit
def kernel(input_sequences, target_sequences, scale_factors, W1, b1, W2, b2,
           W_ih, W_hh, b_ih, b_hh):
    frames_row = input_sequences[:, :, 0].reshape(1, NT)
    ch = input_sequences[:, :, 2:6].reshape(NT, 4)
    q_init = input_sequences[:, :, 0].T.reshape(NT, 1)
    pos_init = jnp.transpose(input_sequences[:, :, 2:6], (1, 0, 2)).reshape(NT, 4)
    q_tgt = target_sequences[:, :, 0].T.reshape(NT, 1)
    scale_row = jnp.concatenate([scale_factors, jnp.ones((4,), F32)]).reshape(1, 8)
    inv_scale = (1.0 / scale_factors).reshape(1, 4)

    mnp = np.zeros((8, 4), np.float32)
    mnp[0, 0] = 1.0
    mnp[1, 1] = 1.0
    mnp[2, 0] = DT
    mnp[2, 2] = 1.0
    mnp[3, 1] = DT
    mnp[3, 3] = 1.0
    enp = np.zeros((HID, 4), np.float32)
    enp[0, 0] = DT * DT
    enp[0, 2] = DT
    enp[1, 1] = DT * DT
    enp[1, 3] = DT
    cnp = np.concatenate([np.eye(4, dtype=np.float32)] * 2, axis=1)

    col_scale = np.ones((1, 4 * HID), np.float32) * 0.5
    col_scale[0, 2 * HID:3 * HID] = 1.0
    hp = jax.lax.Precision.HIGHEST
    w2ih = jnp.dot(W2.T, W_ih.T, precision=hp) * col_scale
    bsum = (jnp.dot(b2, W_ih.T, precision=hp)
            + b_ih + b_hh).reshape(1, 4 * HID) * col_scale

    w1t = W1.T
    a1 = w1t[0:4, :] * inv_scale.reshape(4, 1)
    a2 = w1t[4:8, :] * inv_scale.reshape(4, 1)
    b1m = jnp.dot(jnp.asarray(mnp), a1, precision=hp)
    b2m = jnp.dot(jnp.asarray(mnp), a2, precision=hp)
    g1m = jnp.dot(jnp.asarray(enp), a1, precision=hp)
    g2m = jnp.dot(jnp.asarray(enp), a2, precision=hp)
    whht = W_hh.T * col_scale

    out = pl.pallas_call(
        _rollout_kernel,
        out_shape=jax.ShapeDtypeStruct((NT, 4), F32),
    )(frames_row, ch, q_init, pos_init, q_tgt, scale_row, inv_scale,
      jnp.asarray(mnp), jnp.asarray(enp), jnp.asarray(cnp),
      b1m, b2m, g1m, g2m,
      w1t, b1.reshape(1, HID), w2ih, bsum, whht)

    return out.reshape(T, N, 4).transpose(1, 0, 2)


# speedup vs baseline: 1.1675x; 1.1675x over previous
import numpy as np

import jax
import jax.numpy as jnp
from jax.experimental import pallas as pl

F32 = jnp.float32
N, T = 8, 16
NT = N * T
HID = 128
DT = 0.04
HP = jax.lax.Precision.HIGHEST


def _rollout_kernel(frames_ref, ch_ref, qi_ref, pos_ref, qt_ref, scale_ref,
                    invs8_ref, m_ref, e_ref, c_ref, w1_ref, b1_ref,
                    w2ih_ref, bs_ref, whh_ref, out_ref):
    frames = frames_ref[...].astype(jnp.int32)
    q = jnp.concatenate([qi_ref[...], qt_ref[...]], axis=0)
    mask = (q.astype(jnp.int32) == frames).astype(F32)
    cnt = jnp.sum(mask, axis=1, keepdims=True)
    s = jnp.dot(mask, ch_ref[...], preferred_element_type=F32)

    pos = pos_ref[...]
    pooled0 = pos * cnt[:NT] - s[:NT]
    iseq0 = jnp.concatenate([pos, pooled0], axis=1)

    zeros4 = jnp.zeros((NT, 4), F32)
    cntpat = jnp.concatenate([zeros4 + 1.0, zeros4 + cnt[NT:]], axis=1)
    spat = jnp.concatenate([zeros4, s[NT:]], axis=1)

    w1 = w1_ref[...]
    b1 = b1_ref[...]
    w2ih = w2ih_ref[...]
    bs = bs_ref[...]
    whh = whh_ref[...]
    scale = scale_ref[...]
    invs8 = invs8_ref[...]
    mmat = m_ref[...]
    emat = e_ref[...]
    cmat = c_ref[...]

    def mlp(rows):
        fx = jnp.tanh(jnp.dot(rows, w1, preferred_element_type=F32) + b1)
        return jnp.dot(fx, w2ih, preferred_element_type=F32) + bs

    spow = [jnp.ones((1, 8), F32)]
    for k in range(1, T):
        spow.append(spow[-1] * scale)

    X = {}
    for i in range(T):
        xi = mlp(iseq0[i * N:] * spow[i] if i else iseq0)
        for t in range(T - i):
            X[(i, t)] = xi[t * N:(t + 1) * N]

    H = [None] * T
    C = [None] * T
    P = [None] * T
    for sl in range(3 * (T - 1) + T):
        for i in range(T):
            t = sl - 3 * i
            if t < 0 or t >= T:
                continue
            xt = X[(i, t)]
            g = xt if t == 0 else xt + jnp.dot(
                H[i], whh, preferred_element_type=F32)
            tg = jnp.tanh(g)
            ti = tg[:, 0:128]
            tf = tg[:, 128:256]
            tc = tg[:, 256:384]
            to = tg[:, 384:512]
            cc = (0.5 * ((ti + 1.0) * tc) if t == 0
                  else 0.5 * ((tf + 1.0) * C[i] + (ti + 1.0) * tc))
            hh = (0.5 * (to + 1.0)) * jnp.tanh(cc)
            H[i], C[i] = hh, cc

            if t == T - 1:
                last = (iseq0[NT - N:NT] if i == 0 else P[i - 1]) * scale
                u = (jnp.dot(last, mmat, preferred_element_type=F32,
                             precision=HP)
                     + jnp.dot(hh, emat, preferred_element_type=F32,
                               precision=HP)) * invs8
                out_ref[i * N:(i + 1) * N, :] = u
                if i < T - 1:
                    nr = (jnp.dot(u, cmat, preferred_element_type=F32,
                                  precision=HP)
                          * cntpat[i * N:(i + 1) * N]
                          - spat[i * N:(i + 1) * N])
                    P[i] = nr
                    nlad = T - 1 - i
                    lad = jnp.concatenate(
                        [nr * spow[k] for k in range(nlad)], axis=0)
                    xl = mlp(lad)
                    for k in range(nlad):
                        X[(i + 1 + k, T - 1 - k)] = xl[k * N:(k + 1) * N]


@jax.jit
def kernel(input_sequences, target_sequences, scale_factors, W1, b1, W2, b2,
           W_ih, W_hh, b_ih, b_hh):
    frames_row = input_sequences[:, :, 0].reshape(1, NT)
    ch = input_sequences[:, :, 2:6].reshape(NT, 4)
    q_init = input_sequences[:, :, 0].T.reshape(NT, 1)
    pos_init = jnp.transpose(input_sequences[:, :, 2:6], (1, 0, 2)).reshape(NT, 4)
    q_tgt = target_sequences[:, :, 0].T.reshape(NT, 1)
    scale_row = jnp.concatenate([scale_factors, jnp.ones((4,), F32)]).reshape(1, 8)
    inv_scale = (1.0 / scale_factors).reshape(1, 4)

    mnp = np.zeros((8, 4), np.float32)
    mnp[0, 0] = 1.0
    mnp[1, 1] = 1.0
    mnp[2, 0] = DT
    mnp[2, 2] = 1.0
    mnp[3, 1] = DT
    mnp[3, 3] = 1.0
    enp = np.zeros((HID, 4), np.float32)
    enp[0, 0] = DT * DT
    enp[0, 2] = DT
    enp[1, 1] = DT * DT
    enp[1, 3] = DT
    cnp = np.concatenate([np.eye(4, dtype=np.float32)] * 2, axis=1)

    col_scale = np.ones((1, 4 * HID), np.float32) * 0.5
    col_scale[0, 2 * HID:3 * HID] = 1.0
    hp = jax.lax.Precision.HIGHEST
    w2ih = jnp.dot(W2.T, W_ih.T, precision=hp) * col_scale
    bsum = (jnp.dot(b2, W_ih.T, precision=hp)
            + b_ih + b_hh).reshape(1, 4 * HID) * col_scale
    whht = W_hh.T * col_scale

    out = pl.pallas_call(
        _rollout_kernel,
        out_shape=jax.ShapeDtypeStruct((NT, 4), F32),
    )(frames_row, ch, q_init, pos_init, q_tgt, scale_row, inv_scale,
      jnp.asarray(mnp), jnp.asarray(enp), jnp.asarray(cnp),
      W1.T, b1.reshape(1, HID), w2ih, bsum, whht)

    return out.reshape(T, N, 4).transpose(1, 0, 2)


# speedup vs baseline: 1.3418x; 1.1493x over previous
import numpy as np

import jax
import jax.numpy as jnp
from jax.experimental import pallas as pl

F32 = jnp.float32
N, T = 8, 16
NT = N * T
HID = 128
DT = 0.04
HP = jax.lax.Precision.HIGHEST


def _dotg(a, b, dims, precision=None):
    return jax.lax.dot_general(a, b, dimension_numbers=(dims, ((), ())),
                               preferred_element_type=F32,
                               precision=precision)


def _rollout_kernel(iseq_ref, tgt_ref, scale_ref, w1_ref, b1_ref, w2_ref,
                    b2_ref, wih_ref, bih_ref, bhh_ref, whh_ref,
                    m_ref, e2_ref, c_ref, ptm_ref, pinv_ref, out_ref):
    iseq_cm = iseq_ref[...]
    tgt_cm = tgt_ref[...]
    scale4 = scale_ref[...]
    ones14 = jnp.zeros((1, 4), F32) + 1.0
    scale = jnp.concatenate([scale4, ones14], axis=1)
    invs8 = 1.0 / scale4
    mmat = m_ref[...]
    emat2 = e2_ref[...]
    cmat = c_ref[...]
    ptm = ptm_ref[...]
    pinv = pinv_ref[...]

    w1t = jnp.transpose(w1_ref[...])
    b1 = b1_ref[...]
    w2ih = _dotg(w2_ref[...], wih_ref[...], ((0,), (1,)), HP)
    bs = (_dotg(b2_ref[...], wih_ref[...], ((1,), (1,)), HP)
          + bih_ref[...] + bhh_ref[...])
    whh = whh_ref[...]
    lane = jax.lax.broadcasted_iota(jnp.int32, (1, 4 * HID), 1)
    gsc = jnp.where((lane >= 2 * HID) & (lane < 3 * HID), 1.0, 0.5)

    fcol = jnp.concatenate([iseq_cm[:, 0:1], tgt_cm[:, 0:1]], axis=0)
    q = fcol.astype(jnp.int32)
    frow = jnp.transpose(iseq_cm[:, 0:1]).astype(jnp.int32)
    mask = (q == frow).astype(F32)
    cnt = jnp.sum(mask, axis=1, keepdims=True)
    ch = iseq_cm[:, 2:6]
    s = jnp.dot(mask, ch, preferred_element_type=F32)

    pooled0_cm = ch * cnt[:NT] - s[:NT]
    iseq0_cm = jnp.concatenate([ch, pooled0_cm], axis=1)
    iseq0 = _dotg(ptm, iseq0_cm, ((1,), (0,)), HP)

    zeros4 = jnp.zeros((NT, 4), F32)
    cntpat_cm = jnp.concatenate([zeros4 + 1.0, zeros4 + cnt[NT:]], axis=1)
    spat_cm = jnp.concatenate([zeros4, s[NT:]], axis=1)
    cntpat = _dotg(ptm, cntpat_cm, ((1,), (0,)), HP)
    spat = _dotg(ptm, spat_cm, ((1,), (0,)), HP)

    def mlp(rows):
        fx = jnp.tanh(_dotg(rows, w1t, ((1,), (0,))) + b1)
        return _dotg(fx, w2ih, ((1,), (0,))) + bs

    spow = [jnp.zeros((1, 8), F32) + 1.0]
    for k in range(1, T):
        spow.append(spow[-1] * scale)

    X = {}
    for i in range(T):
        xi = mlp(iseq0[i * N:] * spow[i] if i else iseq0)
        for t in range(T - i):
            X[(i, t)] = xi[t * N:(t + 1) * N]

    H = [None] * T
    C = [None] * T
    P = [None] * T
    U = [None] * T
    for sl in range(3 * (T - 1) + T):
        for i in range(T):
            t = sl - 3 * i
            if t < 0 or t >= T:
                continue
            xt = X[(i, t)]
            g = xt if t == 0 else xt + _dotg(H[i], whh, ((1,), (1,)))
            tg = jnp.tanh(g * gsc)
            ti = tg[:, 0:128]
            tf = tg[:, 128:256]
            tc = tg[:, 256:384]
            to = tg[:, 384:512]
            cc = (0.5 * ((ti + 1.0) * tc) if t == 0
                  else 0.5 * ((tf + 1.0) * C[i] + (ti + 1.0) * tc))
            hh = (0.5 * (to + 1.0)) * jnp.tanh(cc)
            H[i], C[i] = hh, cc

            if t == T - 1:
                last = (iseq0[NT - N:NT] if i == 0 else P[i - 1]) * scale
                u = (_dotg(last, mmat, ((1,), (0,)), HP)
                     + _dotg(hh[:, 0:2], emat2, ((1,), (0,)), HP)) * invs8
                U[i] = u
                if i < T - 1:
                    nr = (_dotg(u, cmat, ((1,), (0,)), HP)
                          * cntpat[i * N:(i + 1) * N]
                          - spat[i * N:(i + 1) * N])
                    P[i] = nr
                    nlad = T - 1 - i
                    lad = jnp.concatenate(
                        [nr * spow[k] for k in range(nlad)], axis=0)
                    xl = mlp(lad)
                    for k in range(nlad):
                        X[(i + 1 + k, T - 1 - k)] = xl[k * N:(k + 1) * N]

    uall = jnp.concatenate(U, axis=0)
    out_ref[...] = _dotg(pinv, uall, ((1,), (0,)), HP)


@jax.jit
def kernel(input_sequences, target_sequences, scale_factors, W1, b1, W2, b2,
           W_ih, W_hh, b_ih, b_hh):
    mnp = np.zeros((8, 4), np.float32)
    mnp[0, 0] = 1.0
    mnp[1, 1] = 1.0
    mnp[2, 0] = DT
    mnp[2, 2] = 1.0
    mnp[3, 1] = DT
    mnp[3, 3] = 1.0
    e2np = np.array([[DT * DT, 0.0, DT, 0.0],
                     [0.0, DT * DT, 0.0, DT]], np.float32)
    cnp = np.concatenate([np.eye(4, dtype=np.float32)] * 2, axis=1)
    ptm = np.zeros((NT, NT), np.float32)
    for b in range(N):
        for t in range(T):
            ptm[t * N + b, b * T + t] = 1.0
    pinv = ptm.T.copy()

    out = pl.pallas_call(
        _rollout_kernel,
        out_shape=jax.ShapeDtypeStruct((NT, 4), F32),
    )(input_sequences.reshape(NT, 6), target_sequences.reshape(NT, 6),
      scale_factors.reshape(1, 4), W1, b1.reshape(1, HID), W2,
      b2.reshape(1, HID), W_ih, b_ih.reshape(1, 4 * HID),
      b_hh.reshape(1, 4 * HID), W_hh,
      jnp.asarray(mnp), jnp.asarray(e2np), jnp.asarray(cnp),
      jnp.asarray(ptm), jnp.asarray(pinv))

    return out.reshape(N, T, 4)


# speedup vs baseline: 1.5044x; 1.1212x over previous
import numpy as np

import jax
import jax.numpy as jnp
from jax.experimental import pallas as pl

F32 = jnp.float32
N, T = 8, 16
NT = N * T
HID = 128
DT = 0.04
HP = jax.lax.Precision.HIGHEST


def _dotg(a, b, dims, precision=None):
    return jax.lax.dot_general(a, b, dimension_numbers=(dims, ((), ())),
                               preferred_element_type=F32,
                               precision=precision)


def _rollout_kernel(iseq_ref, tgt_ref, scale_ref, w1_ref, b1_ref, w2_ref,
                    b2_ref, wih_ref, bih_ref, bhh_ref, whh_ref,
                    m_ref, e2_ref, c_ref, ptm_ref, pinv_ref, out_ref):
    iseq_cm = iseq_ref[...]
    tgt_cm = tgt_ref[...]
    scale4 = scale_ref[...]
    ones14 = jnp.zeros((1, 4), F32) + 1.0
    scale = jnp.concatenate([scale4, ones14], axis=1)
    invs8 = 1.0 / scale4
    mmat = m_ref[...]
    emat2 = e2_ref[...]
    cmat = c_ref[...]
    ptm = ptm_ref[...]
    pinv = pinv_ref[...]

    w1t = jnp.transpose(w1_ref[...])
    b1 = b1_ref[...]
    w2ih = _dotg(w2_ref[...], wih_ref[...], ((0,), (1,)), HP)
    bs = (_dotg(b2_ref[...], wih_ref[...], ((1,), (1,)), HP)
          + bih_ref[...] + bhh_ref[...])
    whh = whh_ref[...]
    lane = jax.lax.broadcasted_iota(jnp.int32, (1, 4 * HID), 1)
    gsc = jnp.where((lane >= 2 * HID) & (lane < 3 * HID), 1.0, 0.5)
    row = jax.lax.broadcasted_iota(jnp.int32, (4 * HID, 1), 0)
    rsc = jnp.where((row >= 2 * HID) & (row < 3 * HID), 1.0, 0.5)
    w2ih = w2ih * gsc
    bs = bs * gsc
    whh = whh * rsc

    fcol = jnp.concatenate([iseq_cm[:, 0:1], tgt_cm[:, 0:1]], axis=0)
    q = fcol.astype(jnp.int32)
    frow = jnp.transpose(iseq_cm[:, 0:1]).astype(jnp.int32)
    mask = (q == frow).astype(F32)
    cnt = jnp.sum(mask, axis=1, keepdims=True)
    ch = iseq_cm[:, 2:6]
    s = jnp.dot(mask, ch, preferred_element_type=F32)

    pooled0_cm = ch * cnt[:NT] - s[:NT]
    iseq0_cm = jnp.concatenate([ch, pooled0_cm], axis=1)
    iseq0 = _dotg(ptm, iseq0_cm, ((1,), (0,)), HP)

    zeros4 = jnp.zeros((NT, 4), F32)
    cntpat_cm = jnp.concatenate([zeros4 + 1.0, zeros4 + cnt[NT:]], axis=1)
    spat_cm = jnp.concatenate([zeros4, s[NT:]], axis=1)
    cntpat = _dotg(ptm, cntpat_cm, ((1,), (0,)), HP)
    spat = _dotg(ptm, spat_cm, ((1,), (0,)), HP)

    def mlp(rows):
        fx = jnp.tanh(_dotg(rows, w1t, ((1,), (0,))) + b1)
        return _dotg(fx, w2ih, ((1,), (0,))) + bs

    spow = [jnp.zeros((1, 8), F32) + 1.0]
    for k in range(1, T):
        spow.append(spow[-1] * scale)

    rows0 = [iseq0[i * N:] * spow[i] if i else iseq0 for i in range(T)]
    x0 = mlp(jnp.concatenate(rows0, axis=0))
    X = {}
    off = 0
    for i in range(T):
        for t in range(T - i):
            X[(i, t)] = x0[off + t * N:off + (t + 1) * N]
        off += (T - i) * N

    SP = 2
    H = [None] * T
    C = [None] * T
    P = [None] * T
    U = [None] * T
    for sl in range(SP * (T - 1) + T):
        for i in range(T):
            t = sl - SP * i
            if t < 0 or t >= T:
                continue
            xt = X[(i, t)]
            g = xt if t == 0 else xt + _dotg(H[i], whh, ((1,), (1,)))
            tg = jnp.tanh(g)
            ti = tg[:, 0:128]
            tf = tg[:, 128:256]
            tc = tg[:, 256:384]
            to = tg[:, 384:512]
            cc = (0.5 * ((ti + 1.0) * tc) if t == 0
                  else 0.5 * ((tf + 1.0) * C[i] + (ti + 1.0) * tc))
            hh = (0.5 * (to + 1.0)) * jnp.tanh(cc)
            H[i], C[i] = hh, cc

            if t == T - 1:
                last = (iseq0[NT - N:NT] if i == 0 else P[i - 1]) * scale
                u = (_dotg(last, mmat, ((1,), (0,)), HP)
                     + _dotg(hh[:, 0:2], emat2, ((1,), (0,)), HP)) * invs8
                U[i] = u
                if i < T - 1:
                    nr = (_dotg(u, cmat, ((1,), (0,)), HP)
                          * cntpat[i * N:(i + 1) * N]
                          - spat[i * N:(i + 1) * N])
                    P[i] = nr
                    nlad = T - 1 - i
                    lad = jnp.concatenate(
                        [nr * spow[k] for k in range(nlad)], axis=0)
                    xl = mlp(lad)
                    for k in range(nlad):
                        X[(i + 1 + k, T - 1 - k)] = xl[k * N:(k + 1) * N]

    uall = jnp.concatenate(U, axis=0)
    out_ref[...] = _dotg(pinv, uall, ((1,), (0,)), HP)


@jax.jit
def kernel(input_sequences, target_sequences, scale_factors, W1, b1, W2, b2,
           W_ih, W_hh, b_ih, b_hh):
    mnp = np.zeros((8, 4), np.float32)
    mnp[0, 0] = 1.0
    mnp[1, 1] = 1.0
    mnp[2, 0] = DT
    mnp[2, 2] = 1.0
    mnp[3, 1] = DT
    mnp[3, 3] = 1.0
    e2np = np.array([[DT * DT, 0.0, DT, 0.0],
                     [0.0, DT * DT, 0.0, DT]], np.float32)
    cnp = np.concatenate([np.eye(4, dtype=np.float32)] * 2, axis=1)
    ptm = np.zeros((NT, NT), np.float32)
    for b in range(N):
        for t in range(T):
            ptm[t * N + b, b * T + t] = 1.0
    pinv = ptm.T.copy()

    out = pl.pallas_call(
        _rollout_kernel,
        out_shape=jax.ShapeDtypeStruct((NT, 4), F32),
    )(input_sequences.reshape(NT, 6), target_sequences.reshape(NT, 6),
      scale_factors.reshape(1, 4), W1, b1.reshape(1, HID), W2,
      b2.reshape(1, HID), W_ih, b_ih.reshape(1, 4 * HID),
      b_hh.reshape(1, 4 * HID), W_hh,
      jnp.asarray(mnp), jnp.asarray(e2np), jnp.asarray(cnp),
      jnp.asarray(ptm), jnp.asarray(pinv))

    return out.reshape(N, T, 4)
